# baseline (speedup 1.0000x reference)
import functools

import jax
import jax.numpy as jnp
import numpy as np
from jax.experimental import pallas as pl
from jax.experimental.pallas import tpu as pltpu

_MAX_CAV = 5
_RECORD_LEN = np.array([5, 3, 4, 2, 5, 4, 3, 2], dtype=np.int32)
_C, _H, _W = 256, 48, 176
_B = len(_RECORD_LEN)
_HW = _H * _W
_CP = _C + 3
_NSLOT = _B * _MAX_CAV

_cum = np.concatenate([[0], np.cumsum(_RECORD_LEN)])
_src_list = []
_valid_list = []
for _b in range(_B):
    for _l in range(_MAX_CAV):
        if _l < _RECORD_LEN[_b]:
            _src_list.append(_cum[_b] + _l)
            _valid_list.append(1)
        else:
            _src_list.append(_cum[_b] + _RECORD_LEN[_b] - 1)
            _valid_list.append(0)
_SRC = np.array(_src_list, dtype=np.int32)
_VALID = np.array(_valid_list, dtype=np.int32)

_MASK = jnp.asarray(_VALID.reshape(_B, _MAX_CAV), dtype=jnp.int32)

_JT = 2816
_NJ = _HW // _JT


def _regroup_kernel(src_ref, valid_ref, x_ref, pe_ref, out_ref):
    s = pl.program_id(0)
    del src_ref
    valid = valid_ref[s]

    @pl.when(valid == 1)
    def _store_feat():
        out_ref[0, :, 0:_C] = jnp.swapaxes(x_ref[0], 0, 1)

    @pl.when(valid == 0)
    def _store_zero():
        out_ref[0, :, 0:_C] = jnp.zeros((_JT, _C), dtype=out_ref.dtype)

    pe_row = pe_ref[s, :]
    out_ref[0, :, _C:_CP] = jnp.broadcast_to(pe_row[None, :], (_JT, 3))


@functools.partial(jax.jit)
def kernel(spatial_features_2d, prior_encoding):
    x2d = spatial_features_2d.reshape(-1, _C, _HW)
    pe2d = prior_encoding.reshape(_NSLOT, 3)

    grid_spec = pltpu.PrefetchScalarGridSpec(
        num_scalar_prefetch=2,
        grid=(_NSLOT, _NJ),
        in_specs=[
            pl.BlockSpec((1, _C, _JT), lambda s, j, src, valid: (src[s], 0, j)),
            pl.BlockSpec((_NSLOT, 3), lambda s, j, src, valid: (0, 0)),
        ],
        out_specs=pl.BlockSpec((1, _JT, _CP), lambda s, j, src, valid: (s, j, 0)),
    )

    out = pl.pallas_call(
        _regroup_kernel,
        grid_spec=grid_spec,
        out_shape=jax.ShapeDtypeStruct((_NSLOT, _HW, _CP), jnp.float32),
        compiler_params=pltpu.CompilerParams(
            dimension_semantics=("arbitrary", "arbitrary"),
        ),
    )(jnp.asarray(_SRC), jnp.asarray(_VALID), x2d, pe2d)

    regroup_feature = out.reshape(_B, _MAX_CAV, _H, _W, _CP)
    return regroup_feature, _MASK


# speedup vs baseline: 1.0007x; 1.0007x over previous
import functools

import jax
import jax.numpy as jnp
import numpy as np
from jax.experimental import pallas as pl
from jax.experimental.pallas import tpu as pltpu

_MAX_CAV = 5
_RECORD_LEN = np.array([5, 3, 4, 2, 5, 4, 3, 2], dtype=np.int32)
_C, _H, _W = 256, 48, 176
_B = len(_RECORD_LEN)
_HW = _H * _W
_CP = _C + 3
_NSLOT = _B * _MAX_CAV

_cum = np.concatenate([[0], np.cumsum(_RECORD_LEN)])
_src_list = []
_valid_list = []
for _b in range(_B):
    for _l in range(_MAX_CAV):
        if _l < _RECORD_LEN[_b]:
            _src_list.append(_cum[_b] + _l)
            _valid_list.append(1)
        else:
            _src_list.append(_cum[_b] + _RECORD_LEN[_b] - 1)
            _valid_list.append(0)
_SRC = np.array(_src_list, dtype=np.int32)
_VALID = np.array(_valid_list, dtype=np.int32)

_MASK = jnp.asarray(_VALID.reshape(_B, _MAX_CAV), dtype=jnp.int32)

_JT = 2816
_NJ = _HW // _JT


def _regroup_kernel(src_ref, valid_ref, x_ref, pe_ref, eye_ref, out_ref):
    s = pl.program_id(0)
    del src_ref
    valid = valid_ref[s]

    @pl.when(valid == 1)
    def _store_feat():
        out_ref[0, :, 0:_C] = jax.lax.dot_general(
            x_ref[0],
            eye_ref[...],
            dimension_numbers=(((0,), (0,)), ((), ())),
            preferred_element_type=jnp.float32,
        )

    @pl.when(valid == 0)
    def _store_zero():
        out_ref[0, :, 0:_C] = jnp.zeros((_JT, _C), dtype=out_ref.dtype)

    pe_row = pe_ref[s, :]
    out_ref[0, :, _C:_CP] = jnp.broadcast_to(pe_row[None, :], (_JT, 3))


@functools.partial(jax.jit)
def kernel(spatial_features_2d, prior_encoding):
    x2d = spatial_features_2d.reshape(-1, _C, _HW)
    pe2d = prior_encoding.reshape(_NSLOT, 3)

    grid_spec = pltpu.PrefetchScalarGridSpec(
        num_scalar_prefetch=2,
        grid=(_NSLOT, _NJ),
        in_specs=[
            pl.BlockSpec((1, _C, _JT), lambda s, j, src, valid: (src[s], 0, j)),
            pl.BlockSpec((_NSLOT, 3), lambda s, j, src, valid: (0, 0)),
            pl.BlockSpec((_C, _C), lambda s, j, src, valid: (0, 0)),
        ],
        out_specs=pl.BlockSpec((1, _JT, _CP), lambda s, j, src, valid: (s, j, 0)),
    )

    out = pl.pallas_call(
        _regroup_kernel,
        grid_spec=grid_spec,
        out_shape=jax.ShapeDtypeStruct((_NSLOT, _HW, _CP), jnp.float32),
        compiler_params=pltpu.CompilerParams(
            dimension_semantics=("arbitrary", "arbitrary"),
        ),
    )(jnp.asarray(_SRC), jnp.asarray(_VALID), x2d, pe2d, jnp.eye(_C, dtype=jnp.float32))

    regroup_feature = out.reshape(_B, _MAX_CAV, _H, _W, _CP)
    return regroup_feature, _MASK


# speedup vs baseline: 1.4813x; 1.4803x over previous
import functools

import jax
import jax.numpy as jnp
import numpy as np
from jax.experimental import pallas as pl
from jax.experimental.pallas import tpu as pltpu

_MAX_CAV = 5
_RECORD_LEN = np.array([5, 3, 4, 2, 5, 4, 3, 2], dtype=np.int32)
_C, _H, _W = 256, 48, 176
_B = len(_RECORD_LEN)
_CP = _C + 3
_NSLOT = _B * _MAX_CAV

_cum = np.concatenate([[0], np.cumsum(_RECORD_LEN)])
_src_list = []
_valid_list = []
for _b in range(_B):
    for _l in range(_MAX_CAV):
        if _l < _RECORD_LEN[_b]:
            _src_list.append(_cum[_b] + _l)
            _valid_list.append(1)
        else:
            _src_list.append(_cum[_b] + _RECORD_LEN[_b] - 1)
            _valid_list.append(0)
_SRC = np.array(_src_list, dtype=np.int32)
_VALID = np.array(_valid_list, dtype=np.int32)

_MASK = jnp.asarray(_VALID.reshape(_B, _MAX_CAV), dtype=jnp.int32)

_HT = 16
_NJ = _H // _HT


def _regroup_kernel(src_ref, valid_ref, x_ref, pe_ref, out_ref):
    s = pl.program_id(0)
    del src_ref
    valid = valid_ref[s]

    @pl.when(valid == 1)
    def _store_feat():
        out_ref[0, 0, :, :, 0:_C] = jnp.transpose(x_ref[0], (1, 2, 0))

    @pl.when(valid == 0)
    def _store_zero():
        out_ref[0, 0, :, :, 0:_C] = jnp.zeros((_HT, _W, _C), dtype=out_ref.dtype)

    pe_row = pe_ref[s, :]
    out_ref[0, 0, :, :, _C:_CP] = jnp.broadcast_to(pe_row[None, None, :], (_HT, _W, 3))


@functools.partial(jax.jit)
def kernel(spatial_features_2d, prior_encoding):
    pe2d = prior_encoding.reshape(_NSLOT, 3)

    grid_spec = pltpu.PrefetchScalarGridSpec(
        num_scalar_prefetch=2,
        grid=(_NSLOT, _NJ),
        in_specs=[
            pl.BlockSpec((1, _C, _HT, _W), lambda s, j, src, valid: (src[s], 0, j, 0)),
            pl.BlockSpec((_NSLOT, 3), lambda s, j, src, valid: (0, 0)),
        ],
        out_specs=pl.BlockSpec(
            (1, 1, _HT, _W, _CP),
            lambda s, j, src, valid: (s // _MAX_CAV, s % _MAX_CAV, j, 0, 0),
        ),
    )

    regroup_feature = pl.pallas_call(
        _regroup_kernel,
        grid_spec=grid_spec,
        out_shape=jax.ShapeDtypeStruct((_B, _MAX_CAV, _H, _W, _CP), jnp.float32),
        compiler_params=pltpu.CompilerParams(
            dimension_semantics=("arbitrary", "arbitrary"),
        ),
    )(jnp.asarray(_SRC), jnp.asarray(_VALID), spatial_features_2d, pe2d)

    return regroup_feature, _MASK


# speedup vs baseline: 2.2534x; 1.5212x over previous
import functools

import jax
import jax.numpy as jnp
import numpy as np
from jax.experimental import pallas as pl
from jax.experimental.pallas import tpu as pltpu

_MAX_CAV = 5
_RECORD_LEN = np.array([5, 3, 4, 2, 5, 4, 3, 2], dtype=np.int32)
_C, _H, _W = 256, 48, 176
_B = len(_RECORD_LEN)
_CP = _C + 3
_NSLOT = _B * _MAX_CAV

_cum = np.concatenate([[0], np.cumsum(_RECORD_LEN)])
_src_list = []
_valid_list = []
for _b in range(_B):
    for _l in range(_MAX_CAV):
        if _l < _RECORD_LEN[_b]:
            _src_list.append(_cum[_b] + _l)
            _valid_list.append(1)
        else:
            _src_list.append(_cum[_b] + _RECORD_LEN[_b] - 1)
            _valid_list.append(0)
_SRC = np.array(_src_list, dtype=np.int32)
_VALID = np.array(_valid_list, dtype=np.int32)

_MASK = jnp.asarray(_VALID.reshape(_B, _MAX_CAV), dtype=jnp.int32)

_HT = 16
_NJ = _H // _HT


def _regroup_kernel(src_ref, valid_ref, x_ref, pe_ref, out_ref, scr_ref, sem):
    s = pl.program_id(0)
    del src_ref
    valid = valid_ref[s]

    @pl.when(valid == 1)
    def _store_feat():
        for h in range(_HT):
            pltpu.make_async_copy(x_ref.at[0, :, h, :], scr_ref.at[h], sem).start()
        for h in range(_HT):
            pltpu.make_async_copy(x_ref.at[0, :, h, :], scr_ref.at[h], sem).wait()
        out_ref[0, 0, :, :, 0:_C] = jnp.transpose(scr_ref[...], (0, 2, 1))

    @pl.when(valid == 0)
    def _store_zero():
        out_ref[0, 0, :, :, 0:_C] = jnp.zeros((_HT, _W, _C), dtype=out_ref.dtype)

    pe_row = pe_ref[s, :]
    out_ref[0, 0, :, :, _C:_CP] = jnp.broadcast_to(pe_row[None, None, :], (_HT, _W, 3))


@functools.partial(jax.jit)
def kernel(spatial_features_2d, prior_encoding):
    pe2d = prior_encoding.reshape(_NSLOT, 3)

    grid_spec = pltpu.PrefetchScalarGridSpec(
        num_scalar_prefetch=2,
        grid=(_NSLOT, _NJ),
        in_specs=[
            pl.BlockSpec((1, _C, _HT, _W), lambda s, j, src, valid: (src[s], 0, j, 0)),
            pl.BlockSpec((_NSLOT, 3), lambda s, j, src, valid: (0, 0)),
        ],
        out_specs=pl.BlockSpec(
            (1, 1, _HT, _W, _CP),
            lambda s, j, src, valid: (s // _MAX_CAV, s % _MAX_CAV, j, 0, 0),
        ),
        scratch_shapes=[
            pltpu.VMEM((_HT, _C, _W), jnp.float32),
            pltpu.SemaphoreType.DMA,
        ],
    )

    regroup_feature = pl.pallas_call(
        _regroup_kernel,
        grid_spec=grid_spec,
        out_shape=jax.ShapeDtypeStruct((_B, _MAX_CAV, _H, _W, _CP), jnp.float32),
        compiler_params=pltpu.CompilerParams(
            dimension_semantics=("arbitrary", "arbitrary"),
        ),
    )(jnp.asarray(_SRC), jnp.asarray(_VALID), spatial_features_2d, pe2d)

    return regroup_feature, _MASK


# speedup vs baseline: 7.0015x; 3.1071x over previous
import functools

import jax
import jax.numpy as jnp
import numpy as np
from jax.experimental import pallas as pl
from jax.experimental.pallas import tpu as pltpu

_MAX_CAV = 5
_RECORD_LEN = np.array([5, 3, 4, 2, 5, 4, 3, 2], dtype=np.int32)
_C, _H, _W = 256, 48, 176
_B = len(_RECORD_LEN)
_CP = _C + 3
_NSLOT = _B * _MAX_CAV

_cum = np.concatenate([[0], np.cumsum(_RECORD_LEN)])
_src_list = []
_valid_list = []
for _b in range(_B):
    for _l in range(_MAX_CAV):
        if _l < _RECORD_LEN[_b]:
            _src_list.append(_cum[_b] + _l)
            _valid_list.append(1)
        else:
            _src_list.append(_cum[_b] + _RECORD_LEN[_b] - 1)
            _valid_list.append(0)
_SRC = np.array(_src_list, dtype=np.int32)
_VALID = np.array(_valid_list, dtype=np.int32)

_MASK = jnp.asarray(_VALID.reshape(_B, _MAX_CAV), dtype=jnp.int32)

_HT = 16
_NJ = _H // _HT


def _regroup_kernel(src_ref, valid_ref, x_ref, pe_ref, out_ref, scr_ref, sem):
    s = pl.program_id(1)
    del src_ref
    valid = valid_ref[s]

    @pl.when(valid == 1)
    def _store_feat():
        scr_ref[...] = jnp.transpose(x_ref[0], (0, 2, 1))
        for h in range(_HT):
            pltpu.make_async_copy(
                scr_ref.at[h], out_ref.at[0, 0, pl.ds(0, _C), h, :], sem
            ).start()
        for h in range(_HT):
            pltpu.make_async_copy(
                scr_ref.at[h], out_ref.at[0, 0, pl.ds(0, _C), h, :], sem
            ).wait()

    @pl.when(valid == 0)
    def _store_zero():
        out_ref[0, 0, 0:_C, :, :] = jnp.zeros((_C, _HT, _W), dtype=out_ref.dtype)

    pe_row = pe_ref[s, :]
    out_ref[0, 0, _C:_CP, :, :] = jnp.broadcast_to(
        pe_row[:, None, None], (3, _HT, _W)
    )


@functools.partial(jax.jit)
def kernel(spatial_features_2d, prior_encoding):
    xt = jnp.transpose(spatial_features_2d, (0, 2, 3, 1))
    pe2d = prior_encoding.reshape(_NSLOT, 3)

    grid_spec = pltpu.PrefetchScalarGridSpec(
        num_scalar_prefetch=2,
        grid=(_NJ, _NSLOT),
        in_specs=[
            pl.BlockSpec((1, _HT, _W, _C), lambda j, s, src, valid: (src[s], j, 0, 0)),
            pl.BlockSpec((_NSLOT, 3), lambda j, s, src, valid: (0, 0)),
        ],
        out_specs=pl.BlockSpec(
            (1, 1, _CP, _HT, _W),
            lambda j, s, src, valid: (s // _MAX_CAV, s % _MAX_CAV, 0, j, 0),
        ),
        scratch_shapes=[
            pltpu.VMEM((_HT, _C, _W), jnp.float32),
            pltpu.SemaphoreType.DMA,
        ],
    )

    out5 = pl.pallas_call(
        _regroup_kernel,
        grid_spec=grid_spec,
        out_shape=jax.ShapeDtypeStruct((_B, _MAX_CAV, _CP, _H, _W), jnp.float32),
        compiler_params=pltpu.CompilerParams(
            dimension_semantics=("arbitrary", "arbitrary"),
        ),
    )(jnp.asarray(_SRC), jnp.asarray(_VALID), xt, pe2d)

    regroup_feature = jnp.transpose(out5, (0, 1, 3, 4, 2))
    return regroup_feature, _MASK
